# baseline (speedup 1.0000x reference)
import functools
import math

import jax
import jax.numpy as jnp
from jax import lax
from jax.experimental import pallas as pl
from jax.experimental.pallas import tpu as pltpu
from jax.experimental.pallas import tpu_sc as plsc

D_MODEL = 64
SCALE = math.sqrt(D_MODEL)

NC = 2
NS = 16
NW = NC * NS
L = 16
VECS = D_MODEL // L

NBUF = 4
PIECES = ((0, 96), (96, 104))
MAXLEN = 104


def _make_lookup(R: int, C: int):
    assert R % NW == 0 and C == sum(p[1] for p in PIECES)
    rows_per_w = R // NW
    n_pieces = rows_per_w * 2
    assert n_pieces % NBUF == 0 and n_pieces // NBUF >= 3

    mesh = plsc.VectorSubcoreMesh(core_axis_name="c", subcore_axis_name="s")

    @functools.partial(
        pl.kernel,
        out_type=jax.ShapeDtypeStruct((R, C, D_MODEL), jnp.float32),
        mesh=mesh,
        compiler_params=pltpu.CompilerParams(use_tc_tiling_on_sc=False),
        scratch_types=[
            pltpu.VMEM((rows_per_w, C), jnp.int32),
            [pltpu.VMEM((MAXLEN, D_MODEL), jnp.float32)] * NBUF,
            [pltpu.VMEM((MAXLEN, D_MODEL), jnp.float32)] * NBUF,
            [pltpu.SemaphoreType.DMA] * NBUF,
            [pltpu.SemaphoreType.DMA] * NBUF,
        ],
    )
    def lookup(x_hbm, table_hbm, out_hbm, idx_v, gbufs, wbufs, gsems, wsems):
        wid = lax.axis_index("s") * NC + lax.axis_index("c")
        base_row = wid * rows_per_w
        pltpu.sync_copy(x_hbm.at[pl.ds(base_row, rows_per_w)], idx_v)

        def start_gather(b, p):
            off, ln = PIECES[b % 2]
            r = p // 2
            pltpu.async_copy(
                table_hbm.at[idx_v.at[r, pl.ds(off, ln)]],
                gbufs[b].at[pl.ds(0, ln)],
                gsems[b],
            )

        def gather_done(b):
            _, ln = PIECES[b % 2]
            pltpu.make_async_copy(
                table_hbm.at[pl.ds(0, ln)], gbufs[b].at[pl.ds(0, ln)], gsems[b]
            ).wait()

        def write_done(b):
            off, ln = PIECES[b % 2]
            pltpu.make_async_copy(
                wbufs[b].at[pl.ds(0, ln)],
                out_hbm.at[0, pl.ds(off, ln)],
                wsems[b],
            ).wait()

        def scale(b):
            _, ln = PIECES[b % 2]

            @plsc.parallel_loop(0, ln, step=1, unroll=4)
            def _row(r2):
                for v in range(VECS):
                    sl = pl.ds(v * L, L)
                    wbufs[b][r2, sl] = gbufs[b][r2, sl] * SCALE

        def start_write(b, p):
            off, ln = PIECES[b % 2]
            r = p // 2
            pltpu.async_copy(
                wbufs[b].at[pl.ds(0, ln)],
                out_hbm.at[base_row + r, pl.ds(off, ln)],
                wsems[b],
            )

        for b in range(NBUF):
            start_gather(b, b)

        @pl.loop(0, n_pieces - NBUF, step=NBUF)
        def _outer(p0):
            for b in range(NBUF):
                p = p0 + b
                gather_done(b)

                @pl.when(p0 > 0)
                def _():
                    write_done(b)

                scale(b)
                start_write(b, p)
                start_gather(b, p + NBUF)

        for b in range(NBUF):
            p = n_pieces - NBUF + b
            gather_done(b)
            write_done(b)
            scale(b)
            start_write(b, p)
        for b in range(NBUF):
            write_done(b)

    return lookup


@jax.jit
def kernel(x, table):
    return _make_lookup(x.shape[0], x.shape[1])(x, table)


# speedup vs baseline: 1.2950x; 1.2950x over previous
import functools
import math

import jax
import jax.numpy as jnp
from jax import lax
from jax.experimental import pallas as pl
from jax.experimental.pallas import tpu as pltpu
from jax.experimental.pallas import tpu_sc as plsc

D_MODEL = 64
SCALE = math.sqrt(D_MODEL)

NC = 2
NS = 16
NW = NC * NS
L = 16
VECS = D_MODEL // L

NBUF = 4
PIECES = ((0, 96), (96, 104))
MAXLEN = 104


def _make_lookup(R: int, C: int):
    assert R % NW == 0 and C == sum(p[1] for p in PIECES)
    rows_per_w = R // NW
    n_pieces = rows_per_w * 2
    assert n_pieces % NBUF == 0 and n_pieces // NBUF >= 3

    mesh = plsc.VectorSubcoreMesh(core_axis_name="c", subcore_axis_name="s")

    @functools.partial(
        pl.kernel,
        out_type=jax.ShapeDtypeStruct((R * C, 2 * D_MODEL), jnp.float32),
        mesh=mesh,
        compiler_params=pltpu.CompilerParams(use_tc_tiling_on_sc=False),
        scratch_types=[
            pltpu.VMEM((rows_per_w, C), jnp.int32),
            [pltpu.VMEM((MAXLEN, D_MODEL), jnp.float32)] * NBUF,
            [pltpu.VMEM((MAXLEN, 2 * D_MODEL), jnp.float32)] * NBUF,
            [pltpu.SemaphoreType.DMA] * NBUF,
            [pltpu.SemaphoreType.DMA] * NBUF,
        ],
    )
    def lookup(x_hbm, table_hbm, out_hbm, idx_v, gbufs, wbufs, gsems, wsems):
        wid = lax.axis_index("s") * NC + lax.axis_index("c")
        base_row = wid * rows_per_w
        pltpu.sync_copy(x_hbm.at[pl.ds(base_row, rows_per_w)], idx_v)

        def start_gather(b, p):
            off, ln = PIECES[b % 2]
            r = p // 2
            pltpu.async_copy(
                table_hbm.at[idx_v.at[r, pl.ds(off, ln)]],
                gbufs[b].at[pl.ds(0, ln)],
                gsems[b],
            )

        def gather_done(b):
            _, ln = PIECES[b % 2]
            pltpu.make_async_copy(
                table_hbm.at[pl.ds(0, ln)], gbufs[b].at[pl.ds(0, ln)], gsems[b]
            ).wait()

        def write_done(b):
            _, ln = PIECES[b % 2]
            pltpu.make_async_copy(
                wbufs[b].at[pl.ds(0, ln)],
                out_hbm.at[pl.ds(0, ln)],
                wsems[b],
            ).wait()

        def scale(b):
            _, ln = PIECES[b % 2]

            @plsc.parallel_loop(0, ln, step=1, unroll=4)
            def _row(r2):
                for v in range(VECS):
                    sl = pl.ds(v * L, L)
                    wbufs[b][r2, sl] = gbufs[b][r2, sl] * SCALE

        def start_write(b, p):
            off, ln = PIECES[b % 2]
            r = p // 2
            pltpu.async_copy(
                wbufs[b].at[pl.ds(0, ln)],
                out_hbm.at[pl.ds((base_row + r) * C + off, ln)],
                wsems[b],
            )

        for b in range(NBUF):
            @pl.loop(0, MAXLEN)
            def _z(r2):
                for v in range(VECS):
                    wbufs[b][r2, pl.ds(D_MODEL + v * L, L)] = jnp.zeros(
                        (L,), jnp.float32
                    )

        for b in range(NBUF):
            start_gather(b, b)

        @pl.loop(0, n_pieces - NBUF, step=NBUF)
        def _outer(p0):
            for b in range(NBUF):
                p = p0 + b
                gather_done(b)

                @pl.when(p0 > 0)
                def _():
                    write_done(b)

                scale(b)
                start_write(b, p)
                start_gather(b, p + NBUF)

        for b in range(NBUF):
            p = n_pieces - NBUF + b
            gather_done(b)
            write_done(b)
            scale(b)
            start_write(b, p)
        for b in range(NBUF):
            write_done(b)

    return lookup


@jax.jit
def kernel(x, table):
    R, C = x.shape
    V, D = table.shape
    t128 = jnp.concatenate([table, jnp.zeros((V, D), table.dtype)], axis=1)
    t2m = t128.reshape(2 * V, D)
    pad_out = _make_lookup(R, C)(x * 2, t2m)
    return pad_out[:, :D_MODEL].reshape(R, C, D_MODEL)


# speedup vs baseline: 1.4307x; 1.1048x over previous
import functools
import math

import jax
import jax.numpy as jnp
from jax import lax
from jax.experimental import pallas as pl
from jax.experimental.pallas import tpu as pltpu
from jax.experimental.pallas import tpu_sc as plsc

D_MODEL = 64
SCALE = math.sqrt(D_MODEL)

NC = 2
NS = 16
NW = NC * NS
L = 16
VECS = D_MODEL // L

NBUF = 4
PIECES = ((0, 96), (96, 104))
MAXLEN = 104


def _make_lookup(R: int, C: int):
    assert R % NW == 0 and C == sum(p[1] for p in PIECES)
    rows_per_w = R // NW
    n_pieces = rows_per_w * 2
    assert n_pieces % NBUF == 0 and n_pieces // NBUF >= 3

    mesh = plsc.VectorSubcoreMesh(core_axis_name="c", subcore_axis_name="s")

    @functools.partial(
        pl.kernel,
        out_type=jax.ShapeDtypeStruct((R * C, 2 * D_MODEL), jnp.float32),
        mesh=mesh,
        compiler_params=pltpu.CompilerParams(use_tc_tiling_on_sc=False),
        scratch_types=[
            pltpu.VMEM((rows_per_w, C), jnp.int32),
            [pltpu.VMEM((MAXLEN, D_MODEL), jnp.float32)] * NBUF,
            [pltpu.VMEM((MAXLEN, D_MODEL), jnp.float32)] * NBUF,
            [pltpu.SemaphoreType.DMA] * NBUF,
            [pltpu.SemaphoreType.DMA] * NBUF,
        ],
    )
    def lookup(x_hbm, table_hbm, out_hbm, idx_v, gbufs, wbufs, gsems, wsems):
        wid = lax.axis_index("s") * NC + lax.axis_index("c")
        base_row = wid * rows_per_w
        pltpu.sync_copy(x_hbm.at[pl.ds(base_row, rows_per_w)], idx_v)

        def start_gather(b, p):
            off, ln = PIECES[b % 2]
            r = p // 2
            pltpu.async_copy(
                table_hbm.at[idx_v.at[r, pl.ds(off, ln)]],
                gbufs[b].at[pl.ds(0, ln)],
                gsems[b],
            )

        def gather_done(b):
            _, ln = PIECES[b % 2]
            pltpu.make_async_copy(
                table_hbm.at[pl.ds(0, ln)], gbufs[b].at[pl.ds(0, ln)], gsems[b]
            ).wait()

        def write_done(b):
            _, ln = PIECES[b % 2]
            pltpu.make_async_copy(
                wbufs[b].at[pl.ds(0, ln)],
                out_hbm.at[pl.ds(0, ln), pl.ds(0, D_MODEL)],
                wsems[b],
            ).wait()

        def scale(b):
            _, ln = PIECES[b % 2]

            @plsc.parallel_loop(0, ln, step=1, unroll=4)
            def _row(r2):
                for v in range(VECS):
                    sl = pl.ds(v * L, L)
                    wbufs[b][r2, sl] = gbufs[b][r2, sl] * SCALE

        def start_write(b, p):
            off, ln = PIECES[b % 2]
            r = p // 2
            pltpu.async_copy(
                wbufs[b].at[pl.ds(0, ln)],
                out_hbm.at[pl.ds((base_row + r) * C + off, ln), pl.ds(0, D_MODEL)],
                wsems[b],
            )

        for b in range(NBUF):
            start_gather(b, b)

        @pl.loop(0, n_pieces - NBUF, step=NBUF)
        def _outer(p0):
            for b in range(NBUF):
                p = p0 + b
                gather_done(b)

                @pl.when(p0 > 0)
                def _():
                    write_done(b)

                scale(b)
                start_write(b, p)
                start_gather(b, p + NBUF)

        for b in range(NBUF):
            p = n_pieces - NBUF + b
            gather_done(b)
            write_done(b)
            scale(b)
            start_write(b, p)
        for b in range(NBUF):
            write_done(b)

    return lookup


@jax.jit
def kernel(x, table):
    R, C = x.shape
    V, D = table.shape
    t128 = jnp.concatenate([table, jnp.zeros((V, D), table.dtype)], axis=1)
    t2m = t128.reshape(2 * V, D)
    pad_out = _make_lookup(R, C)(x * 2, t2m)
    return pad_out[:, :D_MODEL].reshape(R, C, D_MODEL)
